# baseline (speedup 1.0000x reference)
import functools
import jax
import jax.numpy as jnp
from jax import lax
from jax.experimental import pallas as pl
from jax.experimental.pallas import tpu as pltpu
from jax.experimental.pallas import tpu_sc as plsc

B = 262144
C = 16
SIZES = (64, 96, 128)
VOLS = tuple(s * s * s for s in SIZES)
YB = 8
NC = 2
NS = 16
NW = NC * NS
PPW = B // NW
CH = 128
NCHUNK = PPW // CH
L = 16
NG = CH // L


def _tec_kernel(xt, g0, g1, g2, out,
                t0, t1, t2, flag,
                in2, out2, fbuf, coords2, idx2, w2, rows2, acc2,
                sem_i0, sem_i1, sem_o0, sem_o1, sem_c0, sem_c1):
    grids = (g0, g1, g2)
    tabs = (t0, t1, t2)
    cid = lax.axis_index("c")
    sid = lax.axis_index("s")
    wid = sid * NC + cid
    lanes = lax.iota(jnp.int32, L)
    csplat = [jnp.full((L,), c, jnp.int32) for c in range(C)]
    sem_in = (sem_i0, sem_i1)
    sem_out = (sem_o0, sem_o1)

    def handshake(target):
        @pl.when(sid == 0)
        def _():
            fbuf[...] = jnp.full((L,), target, jnp.int32)
            pltpu.sync_copy(fbuf, flag.at[cid])

            def poll(done):
                pltpu.sync_copy(flag.at[1 - cid], fbuf)
                return fbuf[...][0] == target

            lax.while_loop(lambda d: jnp.logical_not(d), poll,
                           jnp.array(False))

        plsc.subcore_barrier()

    handshake(0)

    for g in range(3):
        s = SIZES[g]
        nn = YB * s
        zpt = s // NW
        nbr = s // YB
        npairs = (zpt * nbr) // 2
        grid = grids[g]
        tab = tabs[g]

        def issue_in(bi, buf, s=s, grid=grid):
            z = wid * zpt + bi // nbr
            y0 = (bi % nbr) * YB
            for c in range(C):
                pltpu.async_copy(
                    grid.at[0, c, z, pl.ds(y0, YB), :],
                    in2.at[buf, c, pl.ds(0, YB), pl.ds(0, s)], sem_in[buf])

        def wait_in(buf, s=s, grid=grid):
            for c in range(C):
                pltpu.make_async_copy(
                    grid.at[0, c, 0, pl.ds(0, YB), :],
                    in2.at[buf, c, pl.ds(0, YB), pl.ds(0, s)],
                    sem_in[buf]).wait()

        def transpose(buf, s=s):
            def xbody(i, carry2):
                y = i // (s // L)
                xg = i % (s // L)
                jvec = lanes + (buf * 1024 + y * s + xg * L)
                xs = pl.ds(xg * L, L)
                for c in range(C):
                    v = in2[buf, c, y, xs]
                    plsc.store_scatter(out2, [jvec, csplat[c]], v)
                return carry2

            lax.fori_loop(0, YB * (s // L), xbody, 0)

        def issue_out(bi, buf, s=s, nn=nn, tab=tab):
            z = wid * zpt + bi // nbr
            y0 = (bi % nbr) * YB
            node0 = (z * s + y0) * s
            pltpu.async_copy(
                out2.at[pl.ds(buf * 1024, nn), :],
                tab.at[pl.ds(node0, nn), :], sem_out[buf])

        def wait_out(buf, nn=nn, tab=tab):
            pltpu.make_async_copy(
                out2.at[pl.ds(buf * 1024, nn), :],
                tab.at[pl.ds(0, nn), :], sem_out[buf]).wait()

        def pair_body(i2, carry, npairs=npairs):
            bi0 = 2 * i2

            @pl.when(i2 > 0)
            def _():
                wait_out(0)
                wait_out(1)

            wait_in(0)
            transpose(0)
            issue_out(bi0, 0)

            @pl.when(i2 + 1 < npairs)
            def _():
                issue_in(bi0 + 2, 0)

            wait_in(1)
            transpose(1)
            issue_out(bi0 + 1, 1)

            @pl.when(i2 + 1 < npairs)
            def _():
                issue_in(bi0 + 3, 1)

            return carry

        issue_in(0, 0)
        issue_in(1, 1)
        lax.fori_loop(0, npairs, pair_body, 0)
        wait_out(0)
        wait_out(1)

    handshake(1)

    wbase = wid * PPW
    NP = NCHUNK // 2
    sem_gath = (sem_i0, sem_i1)
    sem_out2 = (sem_o0, sem_o1)
    sem_crd = (sem_c0, sem_c1)

    def fire_coords(cp, sub):
        base = wbase + (cp * 2 + sub) * CH
        pltpu.async_copy(xt.at[:, pl.ds(base, CH)], coords2.at[sub],
                         sem_crd[sub])

    def wait_coords(sub):
        pltpu.make_async_copy(xt.at[:, pl.ds(0, CH)], coords2.at[sub],
                              sem_crd[sub]).wait()

    def compute(g, sub, buf):
        s = SIZES[g]
        scale = 0.5 * (s - 1)
        s2 = s * s
        offs = (0, 1, s, s + 1, s2, s2 + 1, s2 + s, s2 + s + 1)

        def grp_body(i, carry2):
            sl = pl.ds(i * L, L)
            gx = coords2[sub, 0, sl]
            gy = coords2[sub, 1, sl]
            gz = coords2[sub, 2, sl]
            fx = gx * scale + scale
            fy = gy * scale + scale
            fz = gz * scale + scale
            x0 = jnp.minimum(jnp.maximum(fx.astype(jnp.int32), 0), s - 2)
            y0 = jnp.minimum(jnp.maximum(fy.astype(jnp.int32), 0), s - 2)
            z0 = jnp.minimum(jnp.maximum(fz.astype(jnp.int32), 0), s - 2)
            wx1 = fx - x0.astype(jnp.float32)
            wy1 = fy - y0.astype(jnp.float32)
            wz1 = fz - z0.astype(jnp.float32)
            wx0 = 1.0 - wx1
            wy0 = 1.0 - wy1
            wz0 = 1.0 - wz1
            ibase = (z0 * s + y0) * s + x0
            a00 = wz0 * wy0
            a01 = wz0 * wy1
            a10 = wz1 * wy0
            a11 = wz1 * wy1
            ws = (a00 * wx0, a00 * wx1, a01 * wx0, a01 * wx1,
                  a10 * wx0, a10 * wx1, a11 * wx0, a11 * wx1)
            for k in range(8):
                idx2[buf, k, sl] = ibase + offs[k]
                w2[buf, k, sl] = ws[k]
            return carry2

        lax.fori_loop(0, NG, grp_body, 0)

    def fire_gath(g, buf):
        for k in range(8):
            pltpu.async_copy(tabs[g].at[idx2.at[buf, k]],
                             rows2.at[buf, k], sem_gath[buf])

    def wait_gath(buf):
        for k in range(8):
            pltpu.make_async_copy(tabs[0].at[pl.ds(0, CH), :],
                                  rows2.at[buf, k], sem_gath[buf]).wait()

    bsplat = [jnp.full((L,), b, jnp.int32) for b in range(2)]
    ksplat = csplat

    def accumulate(g, sub, buf):
        def acc_body(gi, carry2):
            sl = pl.ds(gi * L, L)
            pvec = lanes + gi * L
            wr = [w2[buf, k, sl] for k in range(8)]
            for c in range(C):
                acc = wr[0] * plsc.load_gather(
                    rows2, [bsplat[buf], ksplat[0], pvec, csplat[c]])
                for k in range(1, 8):
                    acc = acc + wr[k] * plsc.load_gather(
                        rows2, [bsplat[buf], ksplat[k], pvec, csplat[c]])
                plsc.store_scatter(
                    acc2, [bsplat[sub], pvec, csplat[c] + (g * C)], acc)
            return carry2

        lax.fori_loop(0, NG, acc_body, 0)

    def fire_out(cp, sub):
        base = wbase + (cp * 2 + sub) * CH
        pltpu.async_copy(acc2.at[sub], out.at[pl.ds(base, CH), :],
                         sem_out2[sub])

    def wait_out2(sub):
        pltpu.make_async_copy(acc2.at[sub], out.at[pl.ds(0, CH), :],
                              sem_out2[sub]).wait()

    fire_coords(0, 0)
    fire_coords(0, 1)

    def pair_body(cp, carry):
        @pl.when(cp > 0)
        def _():
            wait_gath(0)
            accumulate(1, 1, 0)
        wait_coords(0)
        compute(0, 0, 0)
        fire_gath(0, 0)

        @pl.when(cp > 0)
        def _():
            wait_gath(1)
            accumulate(2, 1, 1)
            fire_out(cp - 1, 1)
        compute(1, 0, 1)
        fire_gath(1, 1)

        @pl.when(cp > 0)
        def _():
            wait_out2(0)
        wait_gath(0)
        accumulate(0, 0, 0)
        compute(2, 0, 0)
        fire_gath(2, 0)

        @pl.when(cp + 1 < NP)
        def _():
            fire_coords(cp + 1, 0)

        wait_gath(1)
        accumulate(1, 0, 1)
        wait_coords(1)
        compute(0, 1, 1)
        fire_gath(0, 1)

        wait_gath(0)
        accumulate(2, 0, 0)
        fire_out(cp, 0)
        compute(1, 1, 0)
        fire_gath(1, 0)

        @pl.when(cp > 0)
        def _():
            wait_out2(1)
        wait_gath(1)
        accumulate(0, 1, 1)
        compute(2, 1, 1)
        fire_gath(2, 1)

        @pl.when(cp + 1 < NP)
        def _():
            fire_coords(cp + 1, 1)

        return carry

    lax.fori_loop(0, NP, pair_body, 0)

    wait_gath(0)
    accumulate(1, 1, 0)
    wait_gath(1)
    accumulate(2, 1, 1)
    fire_out(NP - 1, 1)
    wait_out2(0)
    wait_out2(1)


@jax.jit
def kernel(x, grid0, grid1, grid2):
    xt = x.T
    mesh = plsc.VectorSubcoreMesh(core_axis_name="c", subcore_axis_name="s")
    run = pl.kernel(
        _tec_kernel,
        out_type=jax.ShapeDtypeStruct((B, 3 * C), jnp.float32),
        mesh=mesh,
        scratch_types=[
            pltpu.HBM((VOLS[0], C), jnp.float32),
            pltpu.HBM((VOLS[1], C), jnp.float32),
            pltpu.HBM((VOLS[2], C), jnp.float32),
            pltpu.HBM((NC, L), jnp.int32),
            pltpu.VMEM((2, C, YB, 128), jnp.float32),
            pltpu.VMEM((2 * 1024, C), jnp.float32),
            pltpu.VMEM((L,), jnp.int32),
            pltpu.VMEM((2, 3, CH), jnp.float32),
            pltpu.VMEM((2, 8, CH), jnp.int32),
            pltpu.VMEM((2, 8, CH), jnp.float32),
            pltpu.VMEM((2, 8, CH, C), jnp.float32),
            pltpu.VMEM((2, CH, 3 * C), jnp.float32),
            pltpu.SemaphoreType.DMA,
            pltpu.SemaphoreType.DMA,
            pltpu.SemaphoreType.DMA,
            pltpu.SemaphoreType.DMA,
            pltpu.SemaphoreType.DMA,
            pltpu.SemaphoreType.DMA,
        ],
        compiler_params=pltpu.CompilerParams(
            use_tc_tiling_on_sc=False, needs_layout_passes=False),
    )
    return run(xt, grid0, grid1, grid2)


# speedup vs baseline: 1.5365x; 1.5365x over previous
import functools
import jax
import jax.numpy as jnp
from jax import lax
from jax.experimental import pallas as pl
from jax.experimental.pallas import tpu as pltpu
from jax.experimental.pallas import tpu_sc as plsc

B = 262144
C = 16
SIZES = (64, 96, 128)
VOLS = tuple(s * s * s for s in SIZES)
YB = 8
NC = 2
NS = 16
NW = NC * NS
PPW = B // NW
CH = 128
NCHUNK = PPW // CH
L = 16
NG = CH // L


def _tec_kernel(xt, g0, g1, g2, out,
                t0, t1, t2, flag,
                in2, out2, fbuf, coords2, idx2, w2, rows2, acc2,
                sem_i0, sem_i1, sem_o0, sem_o1, sem_c0, sem_c1):
    grids = (g0, g1, g2)
    tabs = (t0, t1, t2)
    cid = lax.axis_index("c")
    sid = lax.axis_index("s")
    wid = sid * NC + cid
    lanes = lax.iota(jnp.int32, L)
    csplat = [jnp.full((L,), c, jnp.int32) for c in range(C)]
    sem_in = (sem_i0, sem_i1)
    sem_out = (sem_o0, sem_o1)

    def handshake(target):
        @pl.when(sid == 0)
        def _():
            fbuf[...] = jnp.full((L,), target, jnp.int32)
            pltpu.sync_copy(fbuf, flag.at[cid])

            def poll(done):
                pltpu.sync_copy(flag.at[1 - cid], fbuf)
                return fbuf[...][0] == target

            lax.while_loop(lambda d: jnp.logical_not(d), poll,
                           jnp.array(False))

        plsc.subcore_barrier()

    handshake(0)

    for g in range(3):
        s = SIZES[g]
        nn = YB * s
        zpt = s // NW
        nbr = s // YB
        npairs = (zpt * nbr) // 2
        grid = grids[g]
        tab = tabs[g]

        def issue_in(bi, buf, s=s, grid=grid):
            z = wid * zpt + bi // nbr
            y0 = (bi % nbr) * YB
            for c in range(C):
                pltpu.async_copy(
                    grid.at[0, c, z, pl.ds(y0, YB), :],
                    in2.at[buf, c, pl.ds(0, YB), pl.ds(0, s)], sem_in[buf])

        def wait_in(buf, s=s, grid=grid):
            for c in range(C):
                pltpu.make_async_copy(
                    grid.at[0, c, 0, pl.ds(0, YB), :],
                    in2.at[buf, c, pl.ds(0, YB), pl.ds(0, s)],
                    sem_in[buf]).wait()

        def transpose(buf, s=s):
            def xbody(i, carry2):
                y = i // (s // L)
                xg = i % (s // L)
                jvec = lanes + (buf * 1024 + y * s + xg * L)
                xs = pl.ds(xg * L, L)
                for c in range(C):
                    v = in2[buf, c, y, xs]
                    plsc.store_scatter(out2, [jvec, csplat[c]], v)
                return carry2

            lax.fori_loop(0, YB * (s // L), xbody, 0)

        def issue_out(bi, buf, s=s, nn=nn, tab=tab):
            z = wid * zpt + bi // nbr
            y0 = (bi % nbr) * YB
            node0 = (z * s + y0) * s
            pltpu.async_copy(
                out2.at[pl.ds(buf * 1024, nn), :],
                tab.at[pl.ds(node0, nn), :], sem_out[buf])

        def wait_out(buf, nn=nn, tab=tab):
            pltpu.make_async_copy(
                out2.at[pl.ds(buf * 1024, nn), :],
                tab.at[pl.ds(0, nn), :], sem_out[buf]).wait()

        def pair_body(i2, carry, npairs=npairs):
            bi0 = 2 * i2

            @pl.when(i2 > 0)
            def _():
                wait_out(0)
                wait_out(1)

            wait_in(0)
            transpose(0)
            issue_out(bi0, 0)

            @pl.when(i2 + 1 < npairs)
            def _():
                issue_in(bi0 + 2, 0)

            wait_in(1)
            transpose(1)
            issue_out(bi0 + 1, 1)

            @pl.when(i2 + 1 < npairs)
            def _():
                issue_in(bi0 + 3, 1)

            return carry

        issue_in(0, 0)
        issue_in(1, 1)
        lax.fori_loop(0, npairs, pair_body, 0)
        wait_out(0)
        wait_out(1)

    handshake(1)

    wbase = wid * PPW
    NP = NCHUNK // 2
    sem_gath = (sem_i0, sem_i1)
    sem_out2 = (sem_o0, sem_o1)
    sem_crd = (sem_c0, sem_c1)

    def fire_coords(cp, sub):
        base = wbase + (cp * 2 + sub) * CH
        pltpu.async_copy(xt.at[:, pl.ds(base, CH)], coords2.at[sub],
                         sem_crd[sub])

    def wait_coords(sub):
        pltpu.make_async_copy(xt.at[:, pl.ds(0, CH)], coords2.at[sub],
                              sem_crd[sub]).wait()

    def compute(g, sub, buf):
        s = SIZES[g]
        scale = 0.5 * (s - 1)
        s2 = s * s
        offs = (0, 1, s, s + 1, s2, s2 + 1, s2 + s, s2 + s + 1)

        def grp_body(i, carry2):
            sl = pl.ds(i * L, L)
            gx = coords2[sub, 0, sl]
            gy = coords2[sub, 1, sl]
            gz = coords2[sub, 2, sl]
            fx = gx * scale + scale
            fy = gy * scale + scale
            fz = gz * scale + scale
            x0 = jnp.minimum(jnp.maximum(fx.astype(jnp.int32), 0), s - 2)
            y0 = jnp.minimum(jnp.maximum(fy.astype(jnp.int32), 0), s - 2)
            z0 = jnp.minimum(jnp.maximum(fz.astype(jnp.int32), 0), s - 2)
            ibase = (z0 * s + y0) * s + x0
            for k in range(8):
                idx2[buf, k, sl] = ibase + offs[k]
            w2[buf, 0, sl] = fx - x0.astype(jnp.float32)
            w2[buf, 1, sl] = fy - y0.astype(jnp.float32)
            w2[buf, 2, sl] = fz - z0.astype(jnp.float32)
            return carry2

        lax.fori_loop(0, NG, grp_body, 0)

    def fire_gath(g, buf):
        for k in range(8):
            pltpu.async_copy(tabs[g].at[idx2.at[buf, k]],
                             rows2.at[buf, k], sem_gath[buf])

    def wait_gath(buf):
        for k in range(8):
            pltpu.make_async_copy(tabs[0].at[pl.ds(0, CH), :],
                                  rows2.at[buf, k], sem_gath[buf]).wait()

    def accumulate(g, sub, buf):
        def acc_body(gi, carry2):
            off = gi * L
            sl = pl.ds(off, L)
            wxr = w2[buf, 0, sl]
            wyr = w2[buf, 1, sl]
            wzr = w2[buf, 2, sl]
            for j in range(L):
                p = off + j
                wx = wxr[j]
                wy = wyr[j]
                wz = wzr[j]
                r = [rows2[buf, k, p, :] for k in range(8)]
                c00 = r[0] + wx * (r[1] - r[0])
                c01 = r[2] + wx * (r[3] - r[2])
                c10 = r[4] + wx * (r[5] - r[4])
                c11 = r[6] + wx * (r[7] - r[6])
                c0 = c00 + wy * (c01 - c00)
                c1 = c10 + wy * (c11 - c10)
                acc2[sub, p, pl.ds(g * C, C)] = c0 + wz * (c1 - c0)
            return carry2

        lax.fori_loop(0, NG, acc_body, 0)

    def fire_out(cp, sub):
        base = wbase + (cp * 2 + sub) * CH
        pltpu.async_copy(acc2.at[sub], out.at[pl.ds(base, CH), :],
                         sem_out2[sub])

    def wait_out2(sub):
        pltpu.make_async_copy(acc2.at[sub], out.at[pl.ds(0, CH), :],
                              sem_out2[sub]).wait()

    fire_coords(0, 0)
    fire_coords(0, 1)

    def pair_body(cp, carry):
        @pl.when(cp > 0)
        def _():
            wait_gath(0)
            accumulate(1, 1, 0)
        wait_coords(0)
        compute(0, 0, 0)
        fire_gath(0, 0)

        @pl.when(cp > 0)
        def _():
            wait_gath(1)
            accumulate(2, 1, 1)
            fire_out(cp - 1, 1)
        compute(1, 0, 1)
        fire_gath(1, 1)

        @pl.when(cp > 0)
        def _():
            wait_out2(0)
        wait_gath(0)
        accumulate(0, 0, 0)
        compute(2, 0, 0)
        fire_gath(2, 0)

        @pl.when(cp + 1 < NP)
        def _():
            fire_coords(cp + 1, 0)

        wait_gath(1)
        accumulate(1, 0, 1)
        wait_coords(1)
        compute(0, 1, 1)
        fire_gath(0, 1)

        wait_gath(0)
        accumulate(2, 0, 0)
        fire_out(cp, 0)
        compute(1, 1, 0)
        fire_gath(1, 0)

        @pl.when(cp > 0)
        def _():
            wait_out2(1)
        wait_gath(1)
        accumulate(0, 1, 1)
        compute(2, 1, 1)
        fire_gath(2, 1)

        @pl.when(cp + 1 < NP)
        def _():
            fire_coords(cp + 1, 1)

        return carry

    lax.fori_loop(0, NP, pair_body, 0)

    wait_gath(0)
    accumulate(1, 1, 0)
    wait_gath(1)
    accumulate(2, 1, 1)
    fire_out(NP - 1, 1)
    wait_out2(0)
    wait_out2(1)


@jax.jit
def kernel(x, grid0, grid1, grid2):
    xt = x.T
    mesh = plsc.VectorSubcoreMesh(core_axis_name="c", subcore_axis_name="s")
    run = pl.kernel(
        _tec_kernel,
        out_type=jax.ShapeDtypeStruct((B, 3 * C), jnp.float32),
        mesh=mesh,
        scratch_types=[
            pltpu.HBM((VOLS[0], C), jnp.float32),
            pltpu.HBM((VOLS[1], C), jnp.float32),
            pltpu.HBM((VOLS[2], C), jnp.float32),
            pltpu.HBM((NC, L), jnp.int32),
            pltpu.VMEM((2, C, YB, 128), jnp.float32),
            pltpu.VMEM((2 * 1024, C), jnp.float32),
            pltpu.VMEM((L,), jnp.int32),
            pltpu.VMEM((2, 3, CH), jnp.float32),
            pltpu.VMEM((2, 8, CH), jnp.int32),
            pltpu.VMEM((2, 8, CH), jnp.float32),
            pltpu.VMEM((2, 8, CH, C), jnp.float32),
            pltpu.VMEM((2, CH, 3 * C), jnp.float32),
            pltpu.SemaphoreType.DMA,
            pltpu.SemaphoreType.DMA,
            pltpu.SemaphoreType.DMA,
            pltpu.SemaphoreType.DMA,
            pltpu.SemaphoreType.DMA,
            pltpu.SemaphoreType.DMA,
        ],
        compiler_params=pltpu.CompilerParams(
            use_tc_tiling_on_sc=False, needs_layout_passes=False),
    )
    return run(xt, grid0, grid1, grid2)


# speedup vs baseline: 1.5396x; 1.0020x over previous
import functools
import jax
import jax.numpy as jnp
from jax import lax
from jax.experimental import pallas as pl
from jax.experimental.pallas import tpu as pltpu
from jax.experimental.pallas import tpu_sc as plsc

B = 262144
C = 16
SIZES = (64, 96, 128)
VOLS = tuple(s * s * s for s in SIZES)
YB = 8
NC = 2
NS = 16
NW = NC * NS
PPW = B // NW
CH = 128
NCHUNK = PPW // CH
L = 16
NG = CH // L


def _tec_kernel(xt, g0, g1, g2, out,
                t0, t1, t2, flag,
                in2, out2, fbuf, coords2, idx2, w2, rows2, acc2,
                sem_i0, sem_i1, sem_o0, sem_o1, sem_c0, sem_c1):
    grids = (g0, g1, g2)
    tabs = (t0, t1, t2)
    cid = lax.axis_index("c")
    sid = lax.axis_index("s")
    wid = sid * NC + cid
    lanes = lax.iota(jnp.int32, L)
    csplat = [jnp.full((L,), c, jnp.int32) for c in range(C)]
    sem_in = (sem_i0, sem_i1)
    sem_out = (sem_o0, sem_o1)

    def handshake(target):
        @pl.when(sid == 0)
        def _():
            fbuf[...] = jnp.full((L,), target, jnp.int32)
            pltpu.sync_copy(fbuf, flag.at[cid])

            def poll(done):
                pltpu.sync_copy(flag.at[1 - cid], fbuf)
                return fbuf[...][0] == target

            lax.while_loop(lambda d: jnp.logical_not(d), poll,
                           jnp.array(False))

        plsc.subcore_barrier()

    handshake(0)

    for g in range(3):
        s = SIZES[g]
        nn = YB * s
        zpt = s // NW
        nbr = s // YB
        npairs = (zpt * nbr) // 2
        grid = grids[g]
        tab = tabs[g]

        def issue_in(bi, buf, s=s, grid=grid):
            z = wid * zpt + bi // nbr
            y0 = (bi % nbr) * YB
            for c in range(C):
                pltpu.async_copy(
                    grid.at[0, c, z, pl.ds(y0, YB), :],
                    in2.at[buf, c, pl.ds(0, YB), pl.ds(0, s)], sem_in[buf])

        def wait_in(buf, s=s, grid=grid):
            for c in range(C):
                pltpu.make_async_copy(
                    grid.at[0, c, 0, pl.ds(0, YB), :],
                    in2.at[buf, c, pl.ds(0, YB), pl.ds(0, s)],
                    sem_in[buf]).wait()

        def transpose(buf, s=s):
            def ybody(y, carry2):
                jb = buf * 1024 + y * s
                for xg in range(s // L):
                    jvec = lanes + (jb + xg * L)
                    for c in range(C):
                        v = in2[buf, c, y, pl.ds(xg * L, L)]
                        plsc.store_scatter(out2, [jvec, csplat[c]], v)
                return carry2

            lax.fori_loop(0, YB, ybody, 0)

        def issue_out(bi, buf, s=s, nn=nn, tab=tab):
            z = wid * zpt + bi // nbr
            y0 = (bi % nbr) * YB
            node0 = (z * s + y0) * s
            pltpu.async_copy(
                out2.at[pl.ds(buf * 1024, nn), :],
                tab.at[pl.ds(node0, nn), :], sem_out[buf])

        def wait_out(buf, nn=nn, tab=tab):
            pltpu.make_async_copy(
                out2.at[pl.ds(buf * 1024, nn), :],
                tab.at[pl.ds(0, nn), :], sem_out[buf]).wait()

        def pair_body(i2, carry, npairs=npairs):
            bi0 = 2 * i2

            @pl.when(i2 > 0)
            def _():
                wait_out(0)
                wait_out(1)

            wait_in(0)
            transpose(0)
            issue_out(bi0, 0)

            @pl.when(i2 + 1 < npairs)
            def _():
                issue_in(bi0 + 2, 0)

            wait_in(1)
            transpose(1)
            issue_out(bi0 + 1, 1)

            @pl.when(i2 + 1 < npairs)
            def _():
                issue_in(bi0 + 3, 1)

            return carry

        issue_in(0, 0)
        issue_in(1, 1)
        lax.fori_loop(0, npairs, pair_body, 0)
        wait_out(0)
        wait_out(1)

    handshake(1)

    wbase = wid * PPW
    NP = NCHUNK // 2
    sem_gath = (sem_i0, sem_i1)
    sem_out2 = (sem_o0, sem_o1)
    sem_crd = (sem_c0, sem_c1)

    def fire_coords(cp, sub):
        base = wbase + (cp * 2 + sub) * CH
        pltpu.async_copy(xt.at[:, pl.ds(base, CH)], coords2.at[sub],
                         sem_crd[sub])

    def wait_coords(sub):
        pltpu.make_async_copy(xt.at[:, pl.ds(0, CH)], coords2.at[sub],
                              sem_crd[sub]).wait()

    def compute(g, sub, buf):
        s = SIZES[g]
        scale = 0.5 * (s - 1)
        s2 = s * s
        offs = (0, 1, s, s + 1, s2, s2 + 1, s2 + s, s2 + s + 1)

        def grp_body(i, carry2):
            sl = pl.ds(i * L, L)
            gx = coords2[sub, 0, sl]
            gy = coords2[sub, 1, sl]
            gz = coords2[sub, 2, sl]
            fx = gx * scale + scale
            fy = gy * scale + scale
            fz = gz * scale + scale
            x0 = jnp.minimum(jnp.maximum(fx.astype(jnp.int32), 0), s - 2)
            y0 = jnp.minimum(jnp.maximum(fy.astype(jnp.int32), 0), s - 2)
            z0 = jnp.minimum(jnp.maximum(fz.astype(jnp.int32), 0), s - 2)
            ibase = (z0 * s + y0) * s + x0
            for k in range(8):
                idx2[buf, k, sl] = ibase + offs[k]
            w2[buf, 0, sl] = fx - x0.astype(jnp.float32)
            w2[buf, 1, sl] = fy - y0.astype(jnp.float32)
            w2[buf, 2, sl] = fz - z0.astype(jnp.float32)
            return carry2

        lax.fori_loop(0, NG, grp_body, 0)

    def fire_gath(g, buf):
        for k in range(8):
            pltpu.async_copy(tabs[g].at[idx2.at[buf, k]],
                             rows2.at[buf, k], sem_gath[buf])

    def wait_gath(buf):
        for k in range(8):
            pltpu.make_async_copy(tabs[0].at[pl.ds(0, CH), :],
                                  rows2.at[buf, k], sem_gath[buf]).wait()

    def accumulate(g, sub, buf):
        def acc_body(gi, carry2):
            off = gi * L
            sl = pl.ds(off, L)
            wxr = w2[buf, 0, sl]
            wyr = w2[buf, 1, sl]
            wzr = w2[buf, 2, sl]
            for j in range(L):
                p = off + j
                wx = wxr[j]
                wy = wyr[j]
                wz = wzr[j]
                r = [rows2[buf, k, p, :] for k in range(8)]
                c00 = r[0] + wx * (r[1] - r[0])
                c01 = r[2] + wx * (r[3] - r[2])
                c10 = r[4] + wx * (r[5] - r[4])
                c11 = r[6] + wx * (r[7] - r[6])
                c0 = c00 + wy * (c01 - c00)
                c1 = c10 + wy * (c11 - c10)
                acc2[sub, p, pl.ds(g * C, C)] = c0 + wz * (c1 - c0)
            return carry2

        lax.fori_loop(0, NG, acc_body, 0)

    def fire_out(cp, sub):
        base = wbase + (cp * 2 + sub) * CH
        pltpu.async_copy(acc2.at[sub], out.at[pl.ds(base, CH), :],
                         sem_out2[sub])

    def wait_out2(sub):
        pltpu.make_async_copy(acc2.at[sub], out.at[pl.ds(0, CH), :],
                              sem_out2[sub]).wait()

    fire_coords(0, 0)
    fire_coords(0, 1)

    def pair_body(cp, carry):
        @pl.when(cp > 0)
        def _():
            wait_gath(0)
            accumulate(1, 1, 0)
        wait_coords(0)
        compute(0, 0, 0)
        fire_gath(0, 0)

        @pl.when(cp > 0)
        def _():
            wait_gath(1)
            accumulate(2, 1, 1)
            fire_out(cp - 1, 1)
        compute(1, 0, 1)
        fire_gath(1, 1)

        @pl.when(cp > 0)
        def _():
            wait_out2(0)
        wait_gath(0)
        accumulate(0, 0, 0)
        compute(2, 0, 0)
        fire_gath(2, 0)

        @pl.when(cp + 1 < NP)
        def _():
            fire_coords(cp + 1, 0)

        wait_gath(1)
        accumulate(1, 0, 1)
        wait_coords(1)
        compute(0, 1, 1)
        fire_gath(0, 1)

        wait_gath(0)
        accumulate(2, 0, 0)
        fire_out(cp, 0)
        compute(1, 1, 0)
        fire_gath(1, 0)

        @pl.when(cp > 0)
        def _():
            wait_out2(1)
        wait_gath(1)
        accumulate(0, 1, 1)
        compute(2, 1, 1)
        fire_gath(2, 1)

        @pl.when(cp + 1 < NP)
        def _():
            fire_coords(cp + 1, 1)

        return carry

    lax.fori_loop(0, NP, pair_body, 0)

    wait_gath(0)
    accumulate(1, 1, 0)
    wait_gath(1)
    accumulate(2, 1, 1)
    fire_out(NP - 1, 1)
    wait_out2(0)
    wait_out2(1)


@jax.jit
def kernel(x, grid0, grid1, grid2):
    xt = x.T
    mesh = plsc.VectorSubcoreMesh(core_axis_name="c", subcore_axis_name="s")
    run = pl.kernel(
        _tec_kernel,
        out_type=jax.ShapeDtypeStruct((B, 3 * C), jnp.float32),
        mesh=mesh,
        scratch_types=[
            pltpu.HBM((VOLS[0], C), jnp.float32),
            pltpu.HBM((VOLS[1], C), jnp.float32),
            pltpu.HBM((VOLS[2], C), jnp.float32),
            pltpu.HBM((NC, L), jnp.int32),
            pltpu.VMEM((2, C, YB, 128), jnp.float32),
            pltpu.VMEM((2 * 1024, C), jnp.float32),
            pltpu.VMEM((L,), jnp.int32),
            pltpu.VMEM((2, 3, CH), jnp.float32),
            pltpu.VMEM((2, 8, CH), jnp.int32),
            pltpu.VMEM((2, 8, CH), jnp.float32),
            pltpu.VMEM((2, 8, CH, C), jnp.float32),
            pltpu.VMEM((2, CH, 3 * C), jnp.float32),
            pltpu.SemaphoreType.DMA,
            pltpu.SemaphoreType.DMA,
            pltpu.SemaphoreType.DMA,
            pltpu.SemaphoreType.DMA,
            pltpu.SemaphoreType.DMA,
            pltpu.SemaphoreType.DMA,
        ],
        compiler_params=pltpu.CompilerParams(
            use_tc_tiling_on_sc=False, needs_layout_passes=False),
    )
    return run(xt, grid0, grid1, grid2)


# speedup vs baseline: 1.8518x; 1.2028x over previous
import functools
import jax
import jax.numpy as jnp
from jax import lax
from jax.experimental import pallas as pl
from jax.experimental.pallas import tpu as pltpu
from jax.experimental.pallas import tpu_sc as plsc

B = 262144
C = 16
SIZES = (64, 96, 128)
VOLS = tuple(s * s * s for s in SIZES)
YB = 8
NC = 2
NS = 16
NW = NC * NS
PPW = B // NW
CH = 128
NCHUNK = PPW // CH
L = 16
NG = CH // L


def _tec_kernel(xt, g0, g1, g2, out,
                t0, t1, t2, flag,
                in2, out2, fbuf, coords2, idx2, w2, rows2, acc2,
                sem_i0, sem_i1, sem_o0, sem_o1, sem_c0, sem_c1):
    grids = (g0, g1, g2)
    tabs = (t0, t1, t2)
    cid = lax.axis_index("c")
    sid = lax.axis_index("s")
    wid = sid * NC + cid
    lanes = lax.iota(jnp.int32, L)
    csplat = [jnp.full((L,), c, jnp.int32) for c in range(C)]
    sem_in = (sem_i0, sem_i1)
    sem_out = (sem_o0, sem_o1)

    def handshake(target):
        @pl.when(sid == 0)
        def _():
            fbuf[...] = jnp.full((L,), target, jnp.int32)
            pltpu.sync_copy(fbuf, flag.at[cid])

            def poll(done):
                pltpu.sync_copy(flag.at[1 - cid], fbuf)
                return fbuf[...][0] == target

            lax.while_loop(lambda d: jnp.logical_not(d), poll,
                           jnp.array(False))

        plsc.subcore_barrier()

    handshake(0)

    for g in range(3):
        s = SIZES[g]
        nn = YB * s
        zpt = s // NW
        nbr = s // YB
        npairs = (zpt * nbr) // 2
        grid = grids[g]
        tab = tabs[g]

        def issue_in(bi, buf, s=s, grid=grid):
            z = wid * zpt + bi // nbr
            y0 = (bi % nbr) * YB
            for c in range(C):
                pltpu.async_copy(
                    grid.at[0, c, z, pl.ds(y0, YB), :],
                    in2.at[buf, c, pl.ds(0, YB), pl.ds(0, s)], sem_in[buf])

        def wait_in(buf, s=s, grid=grid):
            for c in range(C):
                pltpu.make_async_copy(
                    grid.at[0, c, 0, pl.ds(0, YB), :],
                    in2.at[buf, c, pl.ds(0, YB), pl.ds(0, s)],
                    sem_in[buf]).wait()

        def transpose(buf, s=s):
            def ybody(y, carry2):
                jb = buf * 1024 + y * s
                for xg in range(s // L):
                    jvec = lanes + (jb + xg * L)
                    for c in range(C):
                        v = in2[buf, c, y, pl.ds(xg * L, L)]
                        plsc.store_scatter(out2, [jvec, csplat[c]], v)
                return carry2

            lax.fori_loop(0, YB, ybody, 0)

        def issue_out(bi, buf, s=s, nn=nn, tab=tab):
            z = wid * zpt + bi // nbr
            y0 = (bi % nbr) * YB
            node0 = (z * s + y0) * s
            pltpu.async_copy(
                out2.at[pl.ds(buf * 1024, nn), :],
                tab.at[pl.ds(node0, nn), :], sem_out[buf])

        def wait_out(buf, nn=nn, tab=tab):
            pltpu.make_async_copy(
                out2.at[pl.ds(buf * 1024, nn), :],
                tab.at[pl.ds(0, nn), :], sem_out[buf]).wait()

        def pair_body(i2, carry, npairs=npairs):
            bi0 = 2 * i2

            @pl.when(i2 > 0)
            def _():
                wait_out(0)
                wait_out(1)

            wait_in(0)
            transpose(0)
            issue_out(bi0, 0)

            @pl.when(i2 + 1 < npairs)
            def _():
                issue_in(bi0 + 2, 0)

            wait_in(1)
            transpose(1)
            issue_out(bi0 + 1, 1)

            @pl.when(i2 + 1 < npairs)
            def _():
                issue_in(bi0 + 3, 1)

            return carry

        issue_in(0, 0)
        issue_in(1, 1)
        lax.fori_loop(0, npairs, pair_body, 0)
        wait_out(0)
        wait_out(1)

    handshake(1)

    wbase = wid * PPW
    NP = NCHUNK // 2
    sem_gath = (sem_i0, sem_i1)
    sem_out2 = (sem_o0, sem_o1)
    sem_crd = (sem_c0, sem_c1)

    def fire_coords(cp, sub):
        base = wbase + (cp * 2 + sub) * CH
        pltpu.async_copy(xt.at[:, pl.ds(base, CH)], coords2.at[sub],
                         sem_crd[sub])

    def wait_coords(sub):
        pltpu.make_async_copy(xt.at[:, pl.ds(0, CH)], coords2.at[sub],
                              sem_crd[sub]).wait()

    def compute(g, sub, buf):
        s = SIZES[g]
        scale = 0.5 * (s - 1)
        s2 = s * s
        offs = (0, 1, s, s + 1, s2, s2 + 1, s2 + s, s2 + s + 1)

        def grp_body(i, carry2):
            sl = pl.ds(i * L, L)
            gx = coords2[sub, 0, sl]
            gy = coords2[sub, 1, sl]
            gz = coords2[sub, 2, sl]
            fx = gx * scale + scale
            fy = gy * scale + scale
            fz = gz * scale + scale
            x0 = jnp.minimum(jnp.maximum(fx.astype(jnp.int32), 0), s - 2)
            y0 = jnp.minimum(jnp.maximum(fy.astype(jnp.int32), 0), s - 2)
            z0 = jnp.minimum(jnp.maximum(fz.astype(jnp.int32), 0), s - 2)
            wx1 = fx - x0.astype(jnp.float32)
            wy1 = fy - y0.astype(jnp.float32)
            wz1 = fz - z0.astype(jnp.float32)
            wx0 = 1.0 - wx1
            wy0 = 1.0 - wy1
            wz0 = 1.0 - wz1
            ibase = (z0 * s + y0) * s + x0
            a00 = wz0 * wy0
            a01 = wz0 * wy1
            a10 = wz1 * wy0
            a11 = wz1 * wy1
            ws = (a00 * wx0, a00 * wx1, a01 * wx0, a01 * wx1,
                  a10 * wx0, a10 * wx1, a11 * wx0, a11 * wx1)
            for k in range(8):
                idx2[buf, k, sl] = ibase + offs[k]
                w2[buf, k, sl] = ws[k]
            return carry2

        lax.fori_loop(0, NG, grp_body, 0)

    def fire_gath(g, buf):
        for k in range(8):
            pltpu.async_copy(tabs[g].at[idx2.at[buf, k]],
                             rows2.at[buf, k], sem_gath[buf])

    def wait_gath(buf):
        for k in range(8):
            pltpu.make_async_copy(tabs[0].at[pl.ds(0, CH), :],
                                  rows2.at[buf, k], sem_gath[buf]).wait()

    def accumulate(g, sub, buf):
        def acc_body(gi, carry2):
            off = gi * L
            sl = pl.ds(off, L)
            wr = [w2[buf, k, sl] for k in range(8)]
            for j in range(L):
                p = off + j
                acc = rows2[buf, 0, p, :] * wr[0][j]
                for k in range(1, 8):
                    acc = acc + rows2[buf, k, p, :] * wr[k][j]
                acc2[sub, p, pl.ds(g * C, C)] = acc
            return carry2

        lax.fori_loop(0, NG, acc_body, 0)

    def fire_out(cp, sub):
        base = wbase + (cp * 2 + sub) * CH
        pltpu.async_copy(acc2.at[sub],
                         out.at[pl.ds(base, CH), pl.ds(0, 3 * C)],
                         sem_out2[sub])

    def wait_out2(sub):
        pltpu.make_async_copy(acc2.at[sub],
                              out.at[pl.ds(0, CH), pl.ds(0, 3 * C)],
                              sem_out2[sub]).wait()

    fire_coords(0, 0)
    fire_coords(0, 1)

    def pair_body(cp, carry):
        @pl.when(cp > 0)
        def _():
            wait_gath(0)
            accumulate(1, 1, 0)
        wait_coords(0)
        compute(0, 0, 0)
        fire_gath(0, 0)

        @pl.when(cp > 0)
        def _():
            wait_gath(1)
            accumulate(2, 1, 1)
            fire_out(cp - 1, 1)
        compute(1, 0, 1)
        fire_gath(1, 1)

        @pl.when(cp > 0)
        def _():
            wait_out2(0)
        wait_gath(0)
        accumulate(0, 0, 0)
        compute(2, 0, 0)
        fire_gath(2, 0)

        @pl.when(cp + 1 < NP)
        def _():
            fire_coords(cp + 1, 0)

        wait_gath(1)
        accumulate(1, 0, 1)
        wait_coords(1)
        compute(0, 1, 1)
        fire_gath(0, 1)

        wait_gath(0)
        accumulate(2, 0, 0)
        fire_out(cp, 0)
        compute(1, 1, 0)
        fire_gath(1, 0)

        @pl.when(cp > 0)
        def _():
            wait_out2(1)
        wait_gath(1)
        accumulate(0, 1, 1)
        compute(2, 1, 1)
        fire_gath(2, 1)

        @pl.when(cp + 1 < NP)
        def _():
            fire_coords(cp + 1, 1)

        return carry

    lax.fori_loop(0, NP, pair_body, 0)

    wait_gath(0)
    accumulate(1, 1, 0)
    wait_gath(1)
    accumulate(2, 1, 1)
    fire_out(NP - 1, 1)
    wait_out2(0)
    wait_out2(1)


@jax.jit
def kernel(x, grid0, grid1, grid2):
    xt = x.T
    mesh = plsc.VectorSubcoreMesh(core_axis_name="c", subcore_axis_name="s")
    run = pl.kernel(
        _tec_kernel,
        out_type=jax.ShapeDtypeStruct((B, 128), jnp.float32),
        mesh=mesh,
        scratch_types=[
            pltpu.HBM((VOLS[0], C), jnp.float32),
            pltpu.HBM((VOLS[1], C), jnp.float32),
            pltpu.HBM((VOLS[2], C), jnp.float32),
            pltpu.HBM((NC, L), jnp.int32),
            pltpu.VMEM((2, C, YB, 128), jnp.float32),
            pltpu.VMEM((2 * 1024, C), jnp.float32),
            pltpu.VMEM((L,), jnp.int32),
            pltpu.VMEM((2, 3, CH), jnp.float32),
            pltpu.VMEM((2, 8, CH), jnp.int32),
            pltpu.VMEM((2, 8, CH), jnp.float32),
            pltpu.VMEM((2, 8, CH, C), jnp.float32),
            pltpu.VMEM((2, CH, 3 * C), jnp.float32),
            pltpu.SemaphoreType.DMA,
            pltpu.SemaphoreType.DMA,
            pltpu.SemaphoreType.DMA,
            pltpu.SemaphoreType.DMA,
            pltpu.SemaphoreType.DMA,
            pltpu.SemaphoreType.DMA,
        ],
        compiler_params=pltpu.CompilerParams(
            use_tc_tiling_on_sc=False, needs_layout_passes=False),
    )
    return run(xt, grid0, grid1, grid2)[:, :3 * C]


# speedup vs baseline: 2.2406x; 1.2100x over previous
import functools
import jax
import jax.numpy as jnp
from jax import lax
from jax.experimental import pallas as pl
from jax.experimental.pallas import tpu as pltpu
from jax.experimental.pallas import tpu_sc as plsc

B = 262144
C = 16
SIZES = (64, 96, 128)
VOLS = tuple(s * s * s for s in SIZES)
YB = 8
NC = 2
NS = 16
NW = NC * NS
PPW = B // NW
CH = 128
NCHUNK = PPW // CH
NP = NCHUNK // 2
NPAIRS = tuple((s // NW) * (s // YB) // 2 for s in SIZES)
L = 16
NG = CH // L


def _tec_kernel(xt, g0, g1, g2, out,
                t0, t1, t2, flag,
                in2, out2, fbuf, coords2, idx2, w2, rows2, acc2,
                sb0, sb1, st0, st1, sg0, sg1, sv0, sv1, sc0, sc1):
    grids = (g0, g1, g2)
    tabs = (t0, t1, t2)
    cid = lax.axis_index("c")
    sid = lax.axis_index("s")
    wid = sid * NC + cid
    lanes = lax.iota(jnp.int32, L)
    csplat = [jnp.full((L,), c, jnp.int32) for c in range(C)]
    sem_brick = (sb0, sb1)
    sem_tab = (st0, st1)
    sem_gath = (sg0, sg1)
    sem_vout = (sv0, sv1)
    sem_crd = (sc0, sc1)
    wbase = wid * PPW

    def handshake(target):
        @pl.when(sid == 0)
        def _():
            fbuf[...] = jnp.full((L,), target, jnp.int32)
            pltpu.sync_copy(fbuf, flag.at[cid])

            def poll(done):
                pltpu.sync_copy(flag.at[1 - cid], fbuf)
                return fbuf[...][0] == target

            lax.while_loop(lambda d: jnp.logical_not(d), poll,
                           jnp.array(False))

        plsc.subcore_barrier()

    def brick_ops(g):
        s = SIZES[g]
        nn = YB * s
        zpt = s // NW
        nbr = s // YB
        npb = NPAIRS[g]
        grid = grids[g]
        tab = tabs[g]

        def issue_in(bi, buf):
            z = wid * zpt + bi // nbr
            y0 = (bi % nbr) * YB
            for c in range(C):
                pltpu.async_copy(
                    grid.at[0, c, z, pl.ds(y0, YB), :],
                    in2.at[buf, c, pl.ds(0, YB), pl.ds(0, s)],
                    sem_brick[buf])

        def wait_in(buf):
            for c in range(C):
                pltpu.make_async_copy(
                    grid.at[0, c, 0, pl.ds(0, YB), :],
                    in2.at[buf, c, pl.ds(0, YB), pl.ds(0, s)],
                    sem_brick[buf]).wait()

        def transpose(buf):
            def ybody(y, carry2):
                jb = buf * 1024 + y * s
                for xg in range(s // L):
                    jvec = lanes + (jb + xg * L)
                    for c in range(C):
                        v = in2[buf, c, y, pl.ds(xg * L, L)]
                        plsc.store_scatter(out2, [jvec, csplat[c]], v)
                return carry2

            lax.fori_loop(0, YB, ybody, 0)

        def issue_out(bi, buf):
            z = wid * zpt + bi // nbr
            y0 = (bi % nbr) * YB
            node0 = (z * s + y0) * s
            pltpu.async_copy(
                out2.at[pl.ds(buf * 1024, nn), :],
                tab.at[pl.ds(node0, nn), :], sem_tab[buf])

        def wait_out(buf):
            pltpu.make_async_copy(
                out2.at[pl.ds(buf * 1024, nn), :],
                tab.at[pl.ds(0, nn), :], sem_tab[buf]).wait()

        def prime():
            issue_in(0, 0)
            issue_in(1, 1)

        def pair(i2):
            bi0 = 2 * i2

            @pl.when(i2 > 0)
            def _():
                wait_out(0)
                wait_out(1)

            wait_in(0)
            transpose(0)
            issue_out(bi0, 0)

            @pl.when(i2 + 1 < npb)
            def _():
                issue_in(bi0 + 2, 0)

            wait_in(1)
            transpose(1)
            issue_out(bi0 + 1, 1)

            @pl.when(i2 + 1 < npb)
            def _():
                issue_in(bi0 + 3, 1)

        def drain():
            wait_out(0)
            wait_out(1)

        return prime, pair, drain, npb

    def fire_coords(ck, buf):
        base = wbase + ck * CH
        pltpu.async_copy(xt.at[:, pl.ds(base, CH)], coords2.at[buf],
                         sem_crd[buf])

    def wait_coords(buf):
        pltpu.make_async_copy(xt.at[:, pl.ds(0, CH)], coords2.at[buf],
                              sem_crd[buf]).wait()

    def compute(g, buf):
        s = SIZES[g]
        scale = 0.5 * (s - 1)
        s2 = s * s
        offs = (0, 1, s, s + 1, s2, s2 + 1, s2 + s, s2 + s + 1)

        def grp_body(i, carry2):
            sl = pl.ds(i * L, L)
            gx = coords2[buf, 0, sl]
            gy = coords2[buf, 1, sl]
            gz = coords2[buf, 2, sl]
            fx = gx * scale + scale
            fy = gy * scale + scale
            fz = gz * scale + scale
            x0 = jnp.minimum(jnp.maximum(fx.astype(jnp.int32), 0), s - 2)
            y0 = jnp.minimum(jnp.maximum(fy.astype(jnp.int32), 0), s - 2)
            z0 = jnp.minimum(jnp.maximum(fz.astype(jnp.int32), 0), s - 2)
            wx1 = fx - x0.astype(jnp.float32)
            wy1 = fy - y0.astype(jnp.float32)
            wz1 = fz - z0.astype(jnp.float32)
            wx0 = 1.0 - wx1
            wy0 = 1.0 - wy1
            wz0 = 1.0 - wz1
            ibase = (z0 * s + y0) * s + x0
            a00 = wz0 * wy0
            a01 = wz0 * wy1
            a10 = wz1 * wy0
            a11 = wz1 * wy1
            ws = (a00 * wx0, a00 * wx1, a01 * wx0, a01 * wx1,
                  a10 * wx0, a10 * wx1, a11 * wx0, a11 * wx1)
            for k in range(8):
                idx2[buf, k, sl] = ibase + offs[k]
                w2[buf, k, sl] = ws[k]
            return carry2

        lax.fori_loop(0, NG, grp_body, 0)

    def fire_gath(g, buf):
        for k in range(8):
            pltpu.async_copy(tabs[g].at[idx2.at[buf, k]],
                             rows2.at[buf, k], sem_gath[buf])

    def wait_gath(buf):
        for k in range(8):
            pltpu.make_async_copy(tabs[0].at[pl.ds(0, CH), :],
                                  rows2.at[buf, k], sem_gath[buf]).wait()

    def accumulate(buf):
        def acc_body(gi, carry2):
            off = gi * L
            sl = pl.ds(off, L)
            wr = [w2[buf, k, sl] for k in range(8)]
            for j in range(L):
                p = off + j
                acc = rows2[buf, 0, p, :] * wr[0][j]
                for k in range(1, 8):
                    acc = acc + rows2[buf, k, p, :] * wr[k][j]
                acc2[buf, p, :] = acc
            return carry2

        lax.fori_loop(0, NG, acc_body, 0)

    def fire_vout(g, ck, buf):
        base = wbase + ck * CH
        pltpu.async_copy(acc2.at[buf],
                         out.at[pl.ds(base, CH), pl.ds(g * C, C)],
                         sem_vout[buf])

    def wait_vout(g, buf):
        pltpu.make_async_copy(acc2.at[buf],
                              out.at[pl.ds(0, CH), pl.ds(g * C, C)],
                              sem_vout[buf]).wait()

    def lookup_pass(g, brick_g):
        if brick_g is not None:
            bprime, bpair, bdrain, npb = brick_ops(brick_g)
            bprime()
        fire_coords(0, 0)
        fire_coords(1, 1)

        def pair(cp, carry):
            for sub in (0, 1):
                buf = sub

                @pl.when(cp > 0)
                def _(buf=buf):
                    wait_gath(buf)

                    @pl.when(cp > 1)
                    def _():
                        wait_vout(g, buf)

                    accumulate(buf)
                    fire_vout(g, 2 * (cp - 1) + sub, buf)

                @pl.when(cp < NP)
                def _(buf=buf, sub=sub):
                    wait_coords(buf)
                    compute(g, buf)
                    fire_gath(g, buf)

                    @pl.when(cp + 1 < NP)
                    def _():
                        fire_coords(2 * (cp + 1) + sub, buf)

                if brick_g is not None and sub == 0:
                    @pl.when(cp < npb)
                    def _():
                        bpair(cp)

            return carry

        lax.fori_loop(0, NP + 1, pair, 0)
        wait_vout(g, 0)
        wait_vout(g, 1)
        if brick_g is not None:
            bdrain()

    handshake(0)
    prime0, pair0, drain0, npb0 = brick_ops(0)
    prime0()

    def g0_pairs(i2, carry):
        pair0(i2)
        return carry

    lax.fori_loop(0, npb0, g0_pairs, 0)
    drain0()
    handshake(1)
    lookup_pass(0, 1)
    handshake(2)
    lookup_pass(1, 2)
    handshake(3)
    lookup_pass(2, None)


@jax.jit
def kernel(x, grid0, grid1, grid2):
    xt = x.T
    mesh = plsc.VectorSubcoreMesh(core_axis_name="c", subcore_axis_name="s")
    run = pl.kernel(
        _tec_kernel,
        out_type=jax.ShapeDtypeStruct((B, 128), jnp.float32),
        mesh=mesh,
        scratch_types=[
            pltpu.HBM((VOLS[0], C), jnp.float32),
            pltpu.HBM((VOLS[1], C), jnp.float32),
            pltpu.HBM((VOLS[2], C), jnp.float32),
            pltpu.HBM((NC, L), jnp.int32),
            pltpu.VMEM((2, C, YB, 128), jnp.float32),
            pltpu.VMEM((2 * 1024, C), jnp.float32),
            pltpu.VMEM((L,), jnp.int32),
            pltpu.VMEM((2, 3, CH), jnp.float32),
            pltpu.VMEM((2, 8, CH), jnp.int32),
            pltpu.VMEM((2, 8, CH), jnp.float32),
            pltpu.VMEM((2, 8, CH, C), jnp.float32),
            pltpu.VMEM((2, CH, C), jnp.float32),
            pltpu.SemaphoreType.DMA,
            pltpu.SemaphoreType.DMA,
            pltpu.SemaphoreType.DMA,
            pltpu.SemaphoreType.DMA,
            pltpu.SemaphoreType.DMA,
            pltpu.SemaphoreType.DMA,
            pltpu.SemaphoreType.DMA,
            pltpu.SemaphoreType.DMA,
            pltpu.SemaphoreType.DMA,
            pltpu.SemaphoreType.DMA,
        ],
        compiler_params=pltpu.CompilerParams(
            use_tc_tiling_on_sc=False, needs_layout_passes=False),
    )
    return run(xt, grid0, grid1, grid2)[:, :3 * C]
